# baseline (speedup 1.0000x reference)
import functools

import jax
import jax.numpy as jnp
from jax import lax
from jax.experimental import pallas as pl
from jax.experimental.pallas import tpu as pltpu
from jax.experimental.pallas import tpu_sc as plsc

MAXV = 100000
D = 128
HW = 256
QP = 64
NSET = 4


def _sc_gather_sum(idx, table, batch):
    info = plsc.get_sparse_core_info()
    nw = info.num_cores * info.num_subcores
    per_w = batch // nw

    mesh = plsc.VectorSubcoreMesh(core_axis_name="c", subcore_axis_name="s")

    @functools.partial(
        pl.kernel,
        mesh=mesh,
        out_type=jax.ShapeDtypeStruct((batch, HW, D), jnp.float32),
        scratch_types=[
            pltpu.VMEM((2, 3, HW), jnp.int32),
            pltpu.VMEM((NSET, 3, QP, D), jnp.float32),
            pltpu.SemaphoreType.DMA,
            pltpu.SemaphoreType.DMA,
            pltpu.SemaphoreType.DMA,
            pltpu.SemaphoreType.DMA,
            pltpu.SemaphoreType.DMA,
            pltpu.SemaphoreType.DMA,
            pltpu.SemaphoreType.DMA,
            pltpu.SemaphoreType.DMA,
            pltpu.SemaphoreType.DMA,
            pltpu.SemaphoreType.DMA,
        ],
    )
    def sc_k(idx_hbm, table_hbm, mid_hbm, idx_v, gbuf_v,
             isem0, isem1, gsem0, gsem1, gsem2, gsem3,
             wsem0, wsem1, wsem2, wsem3):
        wid = lax.axis_index("s") * info.num_cores + lax.axis_index("c")
        b0 = wid * per_w
        isems = (isem0, isem1)
        gsems = (gsem0, gsem1, gsem2, gsem3)
        wsems = (wsem0, wsem1, wsem2, wsem3)

        def offsets(m):
            for c in (1, 2):
                for j in range(HW // 16):
                    sl = pl.ds(j * 16, 16)
                    idx_v[m, c, sl] = idx_v[m, c, sl] + c * MAXV

        def fire_gathers(m, q, s):
            for c in range(3):
                pltpu.async_copy(
                    table_hbm.at[idx_v.at[m, c, pl.ds(q * QP, QP)]],
                    gbuf_v.at[s, c], gsems[s])

        def wait_gathers(m, q, s):
            for c in range(3):
                pltpu.make_async_copy(
                    table_hbm.at[idx_v.at[m, c, pl.ds(q * QP, QP)]],
                    gbuf_v.at[s, c], gsems[s]).wait()

        def drain_writeout(s):
            pltpu.make_async_copy(
                gbuf_v.at[s, 0], mid_hbm.at[b0, pl.ds(0, QP)],
                wsems[s]).wait()

        def wait_idx(m):
            pltpu.make_async_copy(
                idx_hbm.at[b0], idx_v.at[m], isems[m]).wait()

        pltpu.sync_copy(idx_hbm.at[b0], idx_v.at[0])
        offsets(0)
        fire_gathers(0, 0, 0)

        def img_pair(t, carry):
            for m in range(2):
                img = 2 * t + m
                b = b0 + img
                if m == 0:
                    pltpu.async_copy(idx_hbm.at[b + 1], idx_v.at[1], isems[1])
                else:
                    @pl.when(t + 1 < per_w // 2)
                    def _():
                        pltpu.async_copy(
                            idx_hbm.at[b + 1], idx_v.at[0], isems[0])

                for q in range(NSET):
                    s = q
                    ns = (q + 1) % NSET
                    if q < 3 and m == 0:
                        @pl.when(t > 0)
                        def _():
                            drain_writeout(ns)
                    else:
                        drain_writeout(ns)
                    if q < 3:
                        fire_gathers(m, q + 1, ns)
                    else:
                        nm = (m + 1) % 2
                        if m == 0:
                            wait_idx(nm)
                            offsets(nm)
                            fire_gathers(nm, 0, ns)
                        else:
                            @pl.when(t + 1 < per_w // 2)
                            def _():
                                wait_idx(nm)
                                offsets(nm)
                                fire_gathers(nm, 0, ns)

                    wait_gathers(m, q, s)

                    def px(p, carry2):
                        for g in range(D // 16):
                            sl = pl.ds(g * 16, 16)
                            gbuf_v[s, 0, p, sl] = (
                                gbuf_v[s, 0, p, sl] + gbuf_v[s, 1, p, sl]
                                + gbuf_v[s, 2, p, sl])
                        return carry2

                    lax.fori_loop(0, QP, px, 0, unroll=2)
                    pltpu.async_copy(
                        gbuf_v.at[s, 0], mid_hbm.at[b, pl.ds(q * QP, QP)],
                        wsems[s])
            return carry

        lax.fori_loop(0, per_w // 2, img_pair, 0)
        drain_writeout(1)
        drain_writeout(2)
        drain_writeout(3)

    return sc_k(idx, table)


def _tc_transpose(mid, batch):
    tb = 32

    def body(m_ref, o_ref):
        o_ref[...] = jnp.transpose(m_ref[...], (0, 2, 1))

    return pl.pallas_call(
        body,
        grid=(batch // tb,),
        in_specs=[pl.BlockSpec((tb, HW, D), lambda i: (i, 0, 0))],
        out_specs=pl.BlockSpec((tb, D, HW), lambda i: (i, 0, 0)),
        out_shape=jax.ShapeDtypeStruct((batch, D, HW), jnp.float32),
    )(mid)


def kernel(inputs, embedding):
    batch, _, h, w = inputs.shape
    idx = inputs.reshape(batch, 3, h * w).astype(jnp.int32)
    nchunk = 2
    bc = batch // nchunk
    outs = []
    for c in range(nchunk):
        mid_c = _sc_gather_sum(idx[c * bc:(c + 1) * bc], embedding, bc)
        outs.append(_tc_transpose(mid_c, bc))
    out = jnp.concatenate(outs, axis=0)
    return out.reshape(batch, D, h, w)


# speedup vs baseline: 2.1933x; 2.1933x over previous
import functools

import jax
import jax.numpy as jnp
from jax import lax
from jax.experimental import pallas as pl
from jax.experimental.pallas import tpu as pltpu
from jax.experimental.pallas import tpu_sc as plsc

MAXV = 100000
D = 128
HW = 256
QP = 64
NSET = 4


def _sc_gather_sum(idx, table, batch):
    info = plsc.get_sparse_core_info()
    nw = info.num_cores * info.num_subcores
    per_w = batch // nw

    mesh = plsc.VectorSubcoreMesh(core_axis_name="c", subcore_axis_name="s")

    @functools.partial(
        pl.kernel,
        mesh=mesh,
        out_type=jax.ShapeDtypeStruct((batch, HW, D), jnp.float32),
        scratch_types=[
            pltpu.VMEM((2, 3, HW), jnp.int32),
            pltpu.VMEM((NSET, 3, QP, D), jnp.float32),
            pltpu.SemaphoreType.DMA,
            pltpu.SemaphoreType.DMA,
            pltpu.SemaphoreType.DMA,
            pltpu.SemaphoreType.DMA,
            pltpu.SemaphoreType.DMA,
            pltpu.SemaphoreType.DMA,
            pltpu.SemaphoreType.DMA,
            pltpu.SemaphoreType.DMA,
            pltpu.SemaphoreType.DMA,
            pltpu.SemaphoreType.DMA,
        ],
    )
    def sc_k(idx_hbm, table_hbm, mid_hbm, idx_v, gbuf_v,
             isem0, isem1, gsem0, gsem1, gsem2, gsem3,
             wsem0, wsem1, wsem2, wsem3):
        wid = lax.axis_index("s") * info.num_cores + lax.axis_index("c")
        b0 = wid * per_w
        isems = (isem0, isem1)
        gsems = (gsem0, gsem1, gsem2, gsem3)
        wsems = (wsem0, wsem1, wsem2, wsem3)

        def offsets(m):
            for c in (1, 2):
                for j in range(HW // 16):
                    sl = pl.ds(j * 16, 16)
                    idx_v[m, c, sl] = idx_v[m, c, sl] + c * MAXV

        def fire_gathers(m, q, s):
            for c in range(3):
                pltpu.async_copy(
                    table_hbm.at[idx_v.at[m, c, pl.ds(q * QP, QP)]],
                    gbuf_v.at[s, c], gsems[s])

        def wait_gathers(m, q, s):
            for c in range(3):
                pltpu.make_async_copy(
                    table_hbm.at[idx_v.at[m, c, pl.ds(q * QP, QP)]],
                    gbuf_v.at[s, c], gsems[s]).wait()

        def drain_writeout(s):
            pltpu.make_async_copy(
                gbuf_v.at[s, 0], mid_hbm.at[b0, pl.ds(0, QP)],
                wsems[s]).wait()

        def wait_idx(m):
            pltpu.make_async_copy(
                idx_hbm.at[b0], idx_v.at[m], isems[m]).wait()

        pltpu.sync_copy(idx_hbm.at[b0], idx_v.at[0])
        offsets(0)
        fire_gathers(0, 0, 0)

        def img_pair(t, carry):
            for m in range(2):
                img = 2 * t + m
                b = b0 + img
                if m == 0:
                    pltpu.async_copy(idx_hbm.at[b + 1], idx_v.at[1], isems[1])
                else:
                    @pl.when(t + 1 < per_w // 2)
                    def _():
                        pltpu.async_copy(
                            idx_hbm.at[b + 1], idx_v.at[0], isems[0])

                for q in range(NSET):
                    s = q
                    ns = (q + 1) % NSET
                    if q < 3 and m == 0:
                        @pl.when(t > 0)
                        def _():
                            drain_writeout(ns)
                    else:
                        drain_writeout(ns)
                    if q < 3:
                        fire_gathers(m, q + 1, ns)
                    else:
                        nm = (m + 1) % 2
                        if m == 0:
                            wait_idx(nm)
                            offsets(nm)
                            fire_gathers(nm, 0, ns)
                        else:
                            @pl.when(t + 1 < per_w // 2)
                            def _():
                                wait_idx(nm)
                                offsets(nm)
                                fire_gathers(nm, 0, ns)

                    wait_gathers(m, q, s)

                    def px(p, carry2):
                        for g in range(D // 16):
                            sl = pl.ds(g * 16, 16)
                            gbuf_v[s, 0, p, sl] = (
                                gbuf_v[s, 0, p, sl] + gbuf_v[s, 1, p, sl]
                                + gbuf_v[s, 2, p, sl])
                        return carry2

                    lax.fori_loop(0, QP, px, 0, unroll=2)
                    pltpu.async_copy(
                        gbuf_v.at[s, 0], mid_hbm.at[b, pl.ds(q * QP, QP)],
                        wsems[s])
            return carry

        lax.fori_loop(0, per_w // 2, img_pair, 0)
        drain_writeout(1)
        drain_writeout(2)
        drain_writeout(3)

    return sc_k(idx, table)


def kernel(inputs, embedding):
    batch, _, h, w = inputs.shape
    idx = inputs.reshape(batch, 3, h * w).astype(jnp.int32)
    mid = _sc_gather_sum(idx, embedding, batch)
    return jnp.transpose(mid.reshape(batch, h, w, D), (0, 3, 1, 2))


# speedup vs baseline: 2.3489x; 1.0709x over previous
import functools

import jax
import jax.numpy as jnp
from jax import lax
from jax.experimental import pallas as pl
from jax.experimental.pallas import tpu as pltpu
from jax.experimental.pallas import tpu_sc as plsc

MAXV = 100000
D = 128
HW = 256
QP = 64
NSET = 4


def _sc_gather_sum(idx, table, batch):
    info = plsc.get_sparse_core_info()
    nw = info.num_cores * info.num_subcores
    per_w = batch // nw

    mesh = plsc.VectorSubcoreMesh(core_axis_name="c", subcore_axis_name="s")

    @functools.partial(
        pl.kernel,
        mesh=mesh,
        out_type=jax.ShapeDtypeStruct((batch, HW, D), jnp.float32),
        scratch_types=[
            pltpu.VMEM((2, 3, HW), jnp.int32),
            pltpu.VMEM((NSET, 3, QP, D), jnp.float32),
            pltpu.SemaphoreType.DMA,
            pltpu.SemaphoreType.DMA,
            pltpu.SemaphoreType.DMA,
            pltpu.SemaphoreType.DMA,
            pltpu.SemaphoreType.DMA,
            pltpu.SemaphoreType.DMA,
            pltpu.SemaphoreType.DMA,
            pltpu.SemaphoreType.DMA,
            pltpu.SemaphoreType.DMA,
            pltpu.SemaphoreType.DMA,
        ],
    )
    def sc_k(idx_hbm, table_hbm, mid_hbm, idx_v, gbuf_v,
             isem0, isem1, gsem0, gsem1, gsem2, gsem3,
             wsem0, wsem1, wsem2, wsem3):
        wid = lax.axis_index("s") * info.num_cores + lax.axis_index("c")
        b0 = wid * per_w
        isems = (isem0, isem1)
        gsems = (gsem0, gsem1, gsem2, gsem3)
        wsems = (wsem0, wsem1, wsem2, wsem3)

        def offsets(m):
            for c in (1, 2):
                for j in range(HW // 16):
                    sl = pl.ds(j * 16, 16)
                    idx_v[m, c, sl] = idx_v[m, c, sl] + c * MAXV

        def fire_gathers(m, q, s):
            for c in range(3):
                pltpu.async_copy(
                    table_hbm.at[idx_v.at[m, c, pl.ds(q * QP, QP)]],
                    gbuf_v.at[s, c], gsems[s])

        def wait_gathers(m, q, s):
            for c in range(3):
                pltpu.make_async_copy(
                    table_hbm.at[idx_v.at[m, c, pl.ds(q * QP, QP)]],
                    gbuf_v.at[s, c], gsems[s]).wait()

        def drain_writeout(s):
            pltpu.make_async_copy(
                gbuf_v.at[s, 0], mid_hbm.at[b0, pl.ds(0, QP)],
                wsems[s]).wait()

        def wait_idx(m):
            pltpu.make_async_copy(
                idx_hbm.at[b0], idx_v.at[m], isems[m]).wait()

        pltpu.sync_copy(idx_hbm.at[b0], idx_v.at[0])
        offsets(0)
        fire_gathers(0, 0, 0)
        fire_gathers(0, 1, 1)

        def img_pair(t, carry):
            for m in range(2):
                img = 2 * t + m
                b = b0 + img
                if m == 0:
                    pltpu.async_copy(idx_hbm.at[b + 1], idx_v.at[1], isems[1])
                else:
                    @pl.when(t + 1 < per_w // 2)
                    def _():
                        pltpu.async_copy(
                            idx_hbm.at[b + 1], idx_v.at[0], isems[0])

                for q in range(NSET):
                    s = q
                    ns = (q + 2) % NSET
                    nm = (m + 1) % 2
                    if q < 2 and m == 0:
                        @pl.when(t > 0)
                        def _():
                            drain_writeout(ns)
                    else:
                        drain_writeout(ns)
                    if q < 2:
                        fire_gathers(m, q + 2, ns)
                    elif m == 0:
                        if q == 2:
                            wait_idx(nm)
                            offsets(nm)
                        fire_gathers(nm, q - 2, ns)
                    else:
                        @pl.when(t + 1 < per_w // 2)
                        def _():
                            if q == 2:
                                wait_idx(nm)
                                offsets(nm)
                            fire_gathers(nm, q - 2, ns)

                    wait_gathers(m, q, s)

                    def px(p, carry2):
                        for g in range(D // 16):
                            sl = pl.ds(g * 16, 16)
                            gbuf_v[s, 0, p, sl] = (
                                gbuf_v[s, 0, p, sl] + gbuf_v[s, 1, p, sl]
                                + gbuf_v[s, 2, p, sl])
                        return carry2

                    lax.fori_loop(0, QP, px, 0, unroll=2)
                    pltpu.async_copy(
                        gbuf_v.at[s, 0], mid_hbm.at[b, pl.ds(q * QP, QP)],
                        wsems[s])
            return carry

        lax.fori_loop(0, per_w // 2, img_pair, 0)
        drain_writeout(2)
        drain_writeout(3)

    return sc_k(idx, table)


def kernel(inputs, embedding):
    batch, _, h, w = inputs.shape
    idx = inputs.reshape(batch, 3, h * w).astype(jnp.int32)
    mid = _sc_gather_sum(idx, embedding, batch)
    return jnp.transpose(mid.reshape(batch, h, w, D), (0, 3, 1, 2))
